# baseline (speedup 1.0000x reference)
import functools

import jax
import jax.numpy as jnp
from jax import lax
from jax.experimental import pallas as pl
from jax.experimental.pallas import tpu as pltpu
from jax.experimental.pallas import tpu_sc as plsc

BATCH = 16384
DIM = 64
NC = 2
NS = 16
NW = NC * NS
ROWS_PER_W = BATCH // NW
CHUNK = 128
NCHUNK = ROWS_PER_W // CHUNK
L = 16


def _sc_partials(uidx, fidx, scores, user_emb, feature_emb):
    mesh = plsc.VectorSubcoreMesh(core_axis_name="c", subcore_axis_name="s")

    @functools.partial(
        pl.kernel,
        mesh=mesh,
        out_type=jax.ShapeDtypeStruct((NW, L), jnp.float32),
        compiler_params=pltpu.CompilerParams(needs_layout_passes=False,
                                             use_tc_tiling_on_sc=False),
        scratch_types=[
            pltpu.VMEM((NCHUNK, CHUNK), jnp.int32),
            pltpu.VMEM((NCHUNK, CHUNK), jnp.int32),
            pltpu.VMEM((ROWS_PER_W, DIM), jnp.float32),
            pltpu.VMEM((ROWS_PER_W, DIM), jnp.float32),
            pltpu.VMEM((ROWS_PER_W,), jnp.float32),
            pltpu.VMEM((L,), jnp.float32),
            pltpu.SemaphoreType.DMA,
        ],
    )
    def k(uidx_hbm, fidx_hbm, score_hbm, uemb_hbm, femb_hbm, out_hbm,
          uidx_v, fidx_v, urows_v, frows_v, score_v, acc_v, sem):
        wid = lax.axis_index("s") * NC + lax.axis_index("c")
        pltpu.sync_copy(uidx_hbm.at[wid], uidx_v)
        pltpu.sync_copy(fidx_hbm.at[wid], fidx_v)
        pltpu.sync_copy(score_hbm.at[wid], score_v)
        copies = []
        for j in range(NCHUNK):
            copies.append(pltpu.async_copy(
                uemb_hbm.at[uidx_v.at[j]],
                urows_v.at[pl.ds(j * CHUNK, CHUNK)], sem))
            copies.append(pltpu.async_copy(
                femb_hbm.at[fidx_v.at[j]],
                frows_v.at[pl.ds(j * CHUNK, CHUNK)], sem))
        for c in copies:
            c.wait()

        lane = lax.iota(jnp.int32, L)
        inv_d = jnp.float32(1.0 / DIM)

        def g_body(g, acc):
            sv = score_v[pl.ds(g * L, L)]
            for k in range(L):
                i = g * L + k
                s = urows_v[i, pl.ds(0, L)] * frows_v[i, pl.ds(0, L)]
                for j in range(1, DIM // L):
                    s = s + (urows_v[i, pl.ds(j * L, L)]
                             * frows_v[i, pl.ds(j * L, L)])
                d = jnp.sum(s) * inv_d - sv[k]
                acc = acc + d * d
            return acc

        acc = lax.fori_loop(0, ROWS_PER_W // L, g_body, jnp.float32(0.0))
        acc_v[...] = jnp.where(lane == 0, acc, 0.0)
        pltpu.sync_copy(acc_v, out_hbm.at[wid])

    return k(uidx, fidx, scores, user_emb, feature_emb)


def _combine(partials):
    def body(p_ref, o_ref):
        s = jnp.sum(p_ref[...])
        o_ref[...] = jnp.full((1, 1), jnp.sqrt(s * (1.0 / BATCH) + 1e-6))

    return pl.pallas_call(
        body,
        out_shape=jax.ShapeDtypeStruct((1, 1), jnp.float32),
    )(partials)


def kernel(user_batch, feature_batch, score_batch, user_emb, feature_emb):
    uidx = user_batch.astype(jnp.int32).reshape(NW, NCHUNK, CHUNK)
    fidx = feature_batch.astype(jnp.int32).reshape(NW, NCHUNK, CHUNK)
    scores = score_batch.astype(jnp.float32).reshape(NW, ROWS_PER_W)
    partials = _sc_partials(uidx, fidx, scores, user_emb, feature_emb)
    return _combine(partials)[0, 0]
